# baseline (speedup 1.0000x reference)
import dataclasses
import functools
import math

import jax
import jax.numpy as jnp
from jax import lax
from jax.experimental import pallas as pl
from jax.experimental.pallas import tpu as pltpu
from jax.experimental.pallas import tpu_sc as plsc

N = 10000
E = 320000
H = 128
FREQ = 256
NUM_RBF = 16
NB = 4
OUT_TOKENS = 33
MAX_DIST = 32.0

NC = 2
NS = 16
NW = NC * NS
EPD = E // NW
NP = 10240
RPS = NP // NS

TILE_E = 10240
E_PAD = NW * TILE_E
CH = 32
SEG = 2560
NSEG = TILE_E // SEG
CPS = SEG // CH

ROWS_D2 = E_PAD // 512
COLS_D2 = 512

NBLK = 640
HHALF = H // 2

_SC_MESH = plsc.VectorSubcoreMesh(
    core_axis_name="c", subcore_axis_name="s", num_cores=NC, num_subcores=NS)


def _sc_params():
    cp = pltpu.CompilerParams()
    if "needs_layout_passes" in pltpu.CompilerParams.__dataclass_fields__:
        cp = dataclasses.replace(cp, needs_layout_passes=False)
    return cp


def _d2_body(px_hbm, py_hbm, pz_hbm, src_hbm, dst_hbm, d2_hbm,
             px, py, pz, sv, dv, d2v):
    cid = lax.axis_index("c")
    sid = lax.axis_index("s")
    wid = sid * NC + cid
    pltpu.sync_copy(px_hbm, px)
    pltpu.sync_copy(py_hbm, py)
    pltpu.sync_copy(pz_hbm, pz)
    base = wid * EPD
    pltpu.sync_copy(src_hbm.at[pl.ds(base, EPD)], sv)
    pltpu.sync_copy(dst_hbm.at[pl.ds(base, EPD)], dv)

    @pl.loop(0, EPD, step=16)
    def _(e):
        s = sv[pl.ds(e, 16)]
        t = dv[pl.ds(e, 16)]
        dx = plsc.load_gather(px, [s]) - plsc.load_gather(px, [t])
        dy = plsc.load_gather(py, [s]) - plsc.load_gather(py, [t])
        dz = plsc.load_gather(pz, [s]) - plsc.load_gather(pz, [t])
        d2v[pl.ds(e, 16)] = dx * dx + dy * dy + dz * dz

    pltpu.sync_copy(d2v, d2_hbm.at[pl.ds(base, EPD)])


def _edge_dist2(px, py, pz, src, dst):
    return pl.kernel(
        _d2_body,
        out_type=jax.ShapeDtypeStruct((E,), jnp.float32),
        mesh=_SC_MESH,
        scratch_types=[
            pltpu.VMEM((N,), jnp.float32),
            pltpu.VMEM((N,), jnp.float32),
            pltpu.VMEM((N,), jnp.float32),
            pltpu.VMEM((EPD,), jnp.int32),
            pltpu.VMEM((EPD,), jnp.int32),
            pltpu.VMEM((EPD,), jnp.float32),
        ],
        compiler_params=_sc_params(),
    )(px, py, pz, src, dst)


def _edge_body(ab_hbm, r_hbm, comb_hbm, z_hbm, p_hbm,
               shared, ci_f, dsi, abv, rv, mv, gs0, gs1, ss0, ss1):
    cid = lax.axis_index("c")
    sid = lax.axis_index("s")
    wid = sid * NC + cid
    base = wid * TILE_E
    pltpu.sync_copy(z_hbm.at[pl.ds(sid * RPS, RPS)],
                    shared.at[pl.ds(sid * RPS, RPS)])
    plsc.subcore_barrier()
    gsems = (gs0, gs1)
    ssems = (ss0, ss1)

    QC = CH // 2

    def start_gathers(ebase, k, p):
        abp = abv.at[p]
        for q in range(4):
            pltpu.async_copy(
                ab_hbm.at[ci_f.at[pl.ds(k * 2 * CH + q * QC, QC)]],
                abp.at[pl.ds(q * QC, QC)], gsems[p])
        pltpu.async_copy(r_hbm.at[pl.ds(ebase + k * CH, CH)], rv.at[p],
                         gsems[p])

    def wait_gathers(ebase, k, p):
        abp = abv.at[p]
        for q in range(4):
            pltpu.make_async_copy(
                ab_hbm.at[ci_f.at[pl.ds(k * 2 * CH + q * QC, QC)]],
                abp.at[pl.ds(q * QC, QC)], gsems[p]).wait()
        pltpu.make_async_copy(r_hbm.at[pl.ds(ebase + k * CH, CH)], rv.at[p],
                              gsems[p]).wait()

    def wait_scatter(p):
        pltpu.make_async_copy(mv.at[p], shared.at[dsi.at[p]],
                              ssems[p]).wait()

    def do_chunk(ebase, k, p):
        wait_gathers(ebase, k, p)

        @pl.when(k >= 2)
        def _():
            wait_scatter(p)

        for i in range(CH // 16):
            dsi[p, pl.ds(i * 16, 16)] = (
                ci_f[pl.ds(k * 2 * CH + CH + i * 16, 16)] - NP)

        abvp, rvp, mvp = abv.at[p], rv.at[p], mv.at[p]

        @plsc.parallel_loop(0, CH, unroll=4)
        def _(c):
            for jj in range(H // 16):
                sl = pl.ds(jj * 16, 16)
                xv = abvp[c, sl] + abvp[c + CH, sl] + rvp[c, sl]
                mvp[c, sl] = xv / (1.0 + jnp.exp(-xv))

        pltpu.async_copy(mv.at[p], shared.at[dsi.at[p]], ssems[p], add=True)

        @pl.when(k + 2 < CPS)
        def _():
            start_gathers(ebase, k + 2, p)

    @pl.loop(0, NSEG)
    def _(s):
        ebase = base + s * SEG
        pltpu.sync_copy(comb_hbm.at[pl.ds(2 * ebase, 2 * SEG)], ci_f)
        start_gathers(ebase, 0, 0)
        start_gathers(ebase, 1, 1)

        @pl.loop(0, CPS, step=2)
        def _(k):
            do_chunk(ebase, k, 0)
            do_chunk(ebase, k + 1, 1)

        wait_scatter(0)
        wait_scatter(1)

    plsc.subcore_barrier()
    pltpu.sync_copy(shared.at[pl.ds(sid * RPS, RPS)],
                    p_hbm.at[cid, pl.ds(sid * RPS, RPS)])


def _edge_messages(ab_flat, r, comb, zeros_n):
    return pl.kernel(
        _edge_body,
        out_type=jax.ShapeDtypeStruct((NC, NP, H), jnp.float32),
        mesh=_SC_MESH,
        scratch_types=[
            pltpu.VMEM_SHARED((NP, H), jnp.float32),
            pltpu.VMEM((2 * SEG,), jnp.int32),
            pltpu.VMEM((2, CH), jnp.int32),
            pltpu.VMEM((2, 2 * CH, H), jnp.float32),
            pltpu.VMEM((2, CH, H), jnp.float32),
            pltpu.VMEM((2, CH, H), jnp.float32),
            pltpu.SemaphoreType.DMA,
            pltpu.SemaphoreType.DMA,
            pltpu.SemaphoreType.DMA,
            pltpu.SemaphoreType.DMA,
        ],
        compiler_params=_sc_params(),
    )(ab_flat, r, comb, zeros_n)


def _time_body(t_ref, f_ref, w1_ref, b1_ref, w2_ref, b2_ref, bm_ref, o_ref):
    t = t_ref[0, 0] * 1000.0
    args = t * f_ref[...]
    tf = jnp.concatenate([jnp.cos(args), jnp.sin(args)], axis=-1)
    tf8 = jnp.broadcast_to(tf, (8, FREQ))
    u = jnp.dot(tf8, w1_ref[...], preferred_element_type=jnp.float32)
    u = u + b1_ref[...]
    u = u / (1.0 + jnp.exp(-u))
    te = jnp.dot(u, w2_ref[...], preferred_element_type=jnp.float32)
    te = te + b2_ref[...]
    o_ref[...] = bm_ref[...] + te


def _time_cvec(t_s, freqs, W1, b1r, W2, b2r, bm8):
    return pl.pallas_call(
        _time_body,
        out_shape=jax.ShapeDtypeStruct((8, H), jnp.float32),
        in_specs=[
            pl.BlockSpec(memory_space=pltpu.SMEM),
            pl.BlockSpec((1, FREQ // 2), lambda: (0, 0)),
            pl.BlockSpec((FREQ, H), lambda: (0, 0)),
            pl.BlockSpec((1, H), lambda: (0, 0)),
            pl.BlockSpec((H, H), lambda: (0, 0)),
            pl.BlockSpec((1, H), lambda: (0, 0)),
            pl.BlockSpec((8, H), lambda: (0, 0)),
        ],
        out_specs=pl.BlockSpec((8, H), lambda: (0, 0)),
    )(t_s, freqs, W1, b1r, W2, b2r, bm8)


def _rbf_body(d2_ref, mu_ref, wr_ref, cv_ref, r0, r1, r2, r3):
    d = jnp.sqrt(d2_ref[0] + 1e-8)
    g = d - mu_ref[...]
    e = jnp.exp(g * g * (-1.0 / 8.0))
    rr = lax.dot_general(e, wr_ref[...], (((0,), (0,)), ((), ())),
                         preferred_element_type=jnp.float32)
    cv = cv_ref[...]
    r0[...] = rr[:, 0 * H:1 * H] + cv[0:1]
    r1[...] = rr[:, 1 * H:2 * H] + cv[1:2]
    r2[...] = rr[:, 2 * H:3 * H] + cv[2:3]
    r3[...] = rr[:, 3 * H:4 * H] + cv[3:4]


def _rbf_project(d2m, mu, wr, cvec):
    rspec = pl.BlockSpec((COLS_D2, H), lambda i: (i, 0))
    return pl.pallas_call(
        _rbf_body,
        grid=(ROWS_D2,),
        out_shape=[jax.ShapeDtypeStruct((E_PAD, H), jnp.float32)] * NB,
        in_specs=[
            pl.BlockSpec((1, 1, COLS_D2), lambda i: (i, 0, 0)),
            pl.BlockSpec((NUM_RBF, 1), lambda i: (0, 0)),
            pl.BlockSpec((NUM_RBF, NB * H), lambda i: (0, 0)),
            pl.BlockSpec((8, H), lambda i: (0, 0)),
        ],
        out_specs=[rspec, rspec, rspec, rspec],
    )(d2m, mu, wr, cvec)


def _prep0_body(x_ref, w_ref, ab_ref):
    ab = jnp.dot(x_ref[...], w_ref[...], preferred_element_type=jnp.float32)
    ab_ref[0] = ab[:, :H]
    ab_ref[1] = ab[:, H:]


def _prep0(x, wsd):
    nspec = pl.BlockSpec((NBLK, H), lambda i: (i, 0))
    abspec = pl.BlockSpec((2, NBLK, H), lambda i: (0, i, 0))
    return pl.pallas_call(
        _prep0_body,
        grid=(NP // NBLK,),
        out_shape=jax.ShapeDtypeStruct((2, NP, H), jnp.float32),
        in_specs=[nspec, pl.BlockSpec((H, 2 * H), lambda i: (0, 0))],
        out_specs=abspec,
    )(x, wsd)


def _update_body(last, h_ref, p_ref, wn1_ref, wn2_ref, bn_ref, wx_ref, bx_ref,
                 *outs):
    h = h_ref[...]
    agg = p_ref[0] + p_ref[1]
    u = (jnp.dot(h, wn1_ref[...], preferred_element_type=jnp.float32)
         + jnp.dot(agg, wn2_ref[...], preferred_element_type=jnp.float32)
         + bn_ref[...])
    hn = h + u / (1.0 + jnp.exp(-u))
    nx = jnp.dot(hn, wx_ref[...], preferred_element_type=jnp.float32)
    if last:
        outs[0][...] = nx + bx_ref[...]
    else:
        outs[0][...] = hn
        outs[1][0] = nx[:, :H]
        outs[1][1] = nx[:, H:]


def _update(h, p, wn1, wn2, bnr, wx, bxr, last):
    nspec = pl.BlockSpec((NBLK, H), lambda i: (i, 0))
    wspec = pl.BlockSpec((H, H), lambda i: (0, 0))
    bspec = pl.BlockSpec((1, H), lambda i: (0, 0))
    if last:
        out_shape = jax.ShapeDtypeStruct((NP, H), jnp.float32)
        out_specs = nspec
        wx_spec = pl.BlockSpec((H, H), lambda i: (0, 0))
    else:
        out_shape = [
            jax.ShapeDtypeStruct((NP, H), jnp.float32),
            jax.ShapeDtypeStruct((2, NP, H), jnp.float32),
        ]
        out_specs = [nspec, pl.BlockSpec((2, NBLK, H), lambda i: (0, i, 0))]
        wx_spec = pl.BlockSpec((H, 2 * H), lambda i: (0, 0))
    return pl.pallas_call(
        functools.partial(_update_body, last),
        grid=(NP // NBLK,),
        out_shape=out_shape,
        in_specs=[
            nspec,
            pl.BlockSpec((NC, NBLK, H), lambda i: (0, i, 0)),
            wspec, wspec, bspec, wx_spec, bspec,
        ],
        out_specs=out_specs,
    )(h, p, wn1, wn2, bnr, wx, bxr)


def kernel(x, pos, timestep, edge_index, W1, b1, W2, b2, Wm, bm, Wn, bn,
           Wout, bout):
    f32 = jnp.float32
    src = edge_index[0]
    dst = edge_index[1]
    px_a = pos[:, 0]
    py_a = pos[:, 1]
    pz_a = pos[:, 2]

    half = FREQ // 2
    freqs = jnp.exp(
        -math.log(10000.0) * jnp.arange(half, dtype=f32) / half
    ).reshape(1, half)
    mu = jnp.linspace(0.0, MAX_DIST, NUM_RBF).astype(f32).reshape(NUM_RBF, 1)
    wr = Wm[:, 2 * H:2 * H + NUM_RBF, :].transpose(1, 0, 2).reshape(
        NUM_RBF, NB * H)
    bm8 = jnp.concatenate([bm, jnp.zeros((8 - NB, H), f32)], axis=0)

    cvec = _time_cvec(timestep.reshape(1, 1), freqs, W1, b1.reshape(1, H),
                      W2, b2.reshape(1, H), bm8)

    d2 = _edge_dist2(px_a, py_a, pz_a, src, dst)
    d2p = jnp.pad(d2, (0, E_PAD - E)).reshape(ROWS_D2, 1, COLS_D2)
    rs = _rbf_project(d2p, mu, wr, cvec)

    srcp = jnp.pad(src, (0, E_PAD - E))
    dstp = jnp.pad(dst, (0, E_PAD - E), constant_values=NP - 1)
    comb = jnp.concatenate(
        [srcp.reshape(-1, CH), dstp.reshape(-1, CH) + NP], axis=1
    ).reshape(-1)

    zeros_n = jnp.zeros((NP, H), f32)
    wsd = [jnp.concatenate([Wm[i, :H, :], Wm[i, H:2 * H, :]], axis=1)
           for i in range(NB)]

    wout_pad = jnp.pad(Wout, ((0, 0), (0, H - OUT_TOKENS)))
    bout_pad = jnp.pad(bout, (0, H - OUT_TOKENS)).reshape(1, H)

    h = jnp.pad(x, ((0, NP - N), (0, 0)))
    ab = _prep0(h, wsd[0])
    for i in range(NB):
        p = _edge_messages(ab.reshape(2 * NP, H), rs[i], comb, zeros_n)
        wn1 = Wn[i, :H, :]
        wn2 = Wn[i, H:, :]
        bnr = bn[i].reshape(1, H)
        if i < NB - 1:
            h, ab = _update(h, p, wn1, wn2, bnr, wsd[i + 1], bnr, last=False)
        else:
            out128 = _update(h, p, wn1, wn2, bnr, wout_pad, bout_pad,
                             last=True)
    return out128[:N, :OUT_TOKENS]


# speedup vs baseline: 1.0933x; 1.0933x over previous
import dataclasses
import functools
import math

import jax
import jax.numpy as jnp
from jax import lax
from jax.experimental import pallas as pl
from jax.experimental.pallas import tpu as pltpu
from jax.experimental.pallas import tpu_sc as plsc

N = 10000
E = 320000
H = 128
FREQ = 256
NUM_RBF = 16
NB = 4
OUT_TOKENS = 33
MAX_DIST = 32.0

NC = 2
NS = 16
NW = NC * NS
EPD = E // NW
NP = 10240
RPS = NP // NS

TILE_E = 10240
E_PAD = NW * TILE_E
CH = 32
SEG = 2560
NSEG = TILE_E // SEG
CPS = SEG // CH

ROWS_D2 = E_PAD // 512
COLS_D2 = 512

NBLK = 640
HHALF = H // 2

_SC_MESH = plsc.VectorSubcoreMesh(
    core_axis_name="c", subcore_axis_name="s", num_cores=NC, num_subcores=NS)


def _sc_params():
    cp = pltpu.CompilerParams()
    if "needs_layout_passes" in pltpu.CompilerParams.__dataclass_fields__:
        cp = dataclasses.replace(cp, needs_layout_passes=False)
    return cp


def _d2_body(px_hbm, py_hbm, pz_hbm, src_hbm, dst_hbm, d2_hbm,
             px, py, pz, sv, dv, d2v):
    cid = lax.axis_index("c")
    sid = lax.axis_index("s")
    wid = sid * NC + cid
    pltpu.sync_copy(px_hbm, px)
    pltpu.sync_copy(py_hbm, py)
    pltpu.sync_copy(pz_hbm, pz)
    base = wid * EPD
    pltpu.sync_copy(src_hbm.at[pl.ds(base, EPD)], sv)
    pltpu.sync_copy(dst_hbm.at[pl.ds(base, EPD)], dv)

    @pl.loop(0, EPD, step=16)
    def _(e):
        s = sv[pl.ds(e, 16)]
        t = dv[pl.ds(e, 16)]
        dx = plsc.load_gather(px, [s]) - plsc.load_gather(px, [t])
        dy = plsc.load_gather(py, [s]) - plsc.load_gather(py, [t])
        dz = plsc.load_gather(pz, [s]) - plsc.load_gather(pz, [t])
        d2v[pl.ds(e, 16)] = dx * dx + dy * dy + dz * dz

    pltpu.sync_copy(d2v, d2_hbm.at[pl.ds(base, EPD)])


def _edge_dist2(px, py, pz, src, dst):
    return pl.kernel(
        _d2_body,
        out_type=jax.ShapeDtypeStruct((E,), jnp.float32),
        mesh=_SC_MESH,
        scratch_types=[
            pltpu.VMEM((N,), jnp.float32),
            pltpu.VMEM((N,), jnp.float32),
            pltpu.VMEM((N,), jnp.float32),
            pltpu.VMEM((EPD,), jnp.int32),
            pltpu.VMEM((EPD,), jnp.int32),
            pltpu.VMEM((EPD,), jnp.float32),
        ],
        compiler_params=_sc_params(),
    )(px, py, pz, src, dst)


def _edge_body(a_hbm, b_hbm, r_hbm, src_hbm, dst_hbm, z_hbm, p_hbm,
               shared, si_f, di_f, dsi, av, bv, rv, mv, gs0, gs1, ss0, ss1):
    cid = lax.axis_index("c")
    sid = lax.axis_index("s")
    wid = sid * NC + cid
    base = wid * TILE_E
    pltpu.sync_copy(z_hbm.at[pl.ds(sid * RPS, RPS)],
                    shared.at[pl.ds(sid * RPS, RPS)])
    plsc.subcore_barrier()
    gsems = (gs0, gs1)
    ssems = (ss0, ss1)

    def start_gathers(ebase, k, p):
        pltpu.async_copy(a_hbm.at[si_f.at[pl.ds(k * CH, CH)]], av.at[p],
                         gsems[p])
        pltpu.async_copy(b_hbm.at[di_f.at[pl.ds(k * CH, CH)]], bv.at[p],
                         gsems[p])
        pltpu.async_copy(r_hbm.at[pl.ds(ebase + k * CH, CH)], rv.at[p],
                         gsems[p])

    def wait_gathers(ebase, k, p):
        pltpu.make_async_copy(a_hbm.at[si_f.at[pl.ds(k * CH, CH)]], av.at[p],
                              gsems[p]).wait()
        pltpu.make_async_copy(b_hbm.at[di_f.at[pl.ds(k * CH, CH)]], bv.at[p],
                              gsems[p]).wait()
        pltpu.make_async_copy(r_hbm.at[pl.ds(ebase + k * CH, CH)], rv.at[p],
                              gsems[p]).wait()

    def wait_scatter(p):
        pltpu.make_async_copy(mv.at[p], shared.at[dsi.at[p]],
                              ssems[p]).wait()

    def do_chunk(ebase, k, p):
        wait_gathers(ebase, k, p)

        @pl.when(k >= 2)
        def _():
            wait_scatter(p)

        for i in range(CH // 16):
            dsi[p, pl.ds(i * 16, 16)] = di_f[pl.ds(k * CH + i * 16, 16)]

        avp, bvp, rvp, mvp = av.at[p], bv.at[p], rv.at[p], mv.at[p]

        @plsc.parallel_loop(0, CH, unroll=4)
        def _(c):
            for jj in range(H // 16):
                sl = pl.ds(jj * 16, 16)
                xv = avp[c, sl] + bvp[c, sl] + rvp[c, sl]
                mvp[c, sl] = xv / (1.0 + jnp.exp(-xv))

        pltpu.async_copy(mv.at[p], shared.at[dsi.at[p]], ssems[p], add=True)

        @pl.when(k + 2 < CPS)
        def _():
            start_gathers(ebase, k + 2, p)

    @pl.loop(0, NSEG)
    def _(s):
        ebase = base + s * SEG
        pltpu.sync_copy(src_hbm.at[pl.ds(ebase, SEG)], si_f)
        pltpu.sync_copy(dst_hbm.at[pl.ds(ebase, SEG)], di_f)
        start_gathers(ebase, 0, 0)
        start_gathers(ebase, 1, 1)

        @pl.loop(0, CPS, step=2)
        def _(k):
            do_chunk(ebase, k, 0)
            do_chunk(ebase, k + 1, 1)

        wait_scatter(0)
        wait_scatter(1)

    plsc.subcore_barrier()
    pltpu.sync_copy(shared.at[pl.ds(sid * RPS, RPS)],
                    p_hbm.at[cid, pl.ds(sid * RPS, RPS)])


def _edge_messages(a, b, r, srcp, dstp, zeros_n):
    return pl.kernel(
        _edge_body,
        out_type=jax.ShapeDtypeStruct((NC, NP, H), jnp.float32),
        mesh=_SC_MESH,
        scratch_types=[
            pltpu.VMEM_SHARED((NP, H), jnp.float32),
            pltpu.VMEM((SEG,), jnp.int32),
            pltpu.VMEM((SEG,), jnp.int32),
            pltpu.VMEM((2, CH), jnp.int32),
            pltpu.VMEM((2, CH, H), jnp.float32),
            pltpu.VMEM((2, CH, H), jnp.float32),
            pltpu.VMEM((2, CH, H), jnp.float32),
            pltpu.VMEM((2, CH, H), jnp.float32),
            pltpu.SemaphoreType.DMA,
            pltpu.SemaphoreType.DMA,
            pltpu.SemaphoreType.DMA,
            pltpu.SemaphoreType.DMA,
        ],
        compiler_params=_sc_params(),
    )(a, b, r, srcp, dstp, zeros_n)


def _time_body(t_ref, f_ref, w1_ref, b1_ref, w2_ref, b2_ref, bm_ref, o_ref):
    t = t_ref[0, 0] * 1000.0
    args = t * f_ref[...]
    tf = jnp.concatenate([jnp.cos(args), jnp.sin(args)], axis=-1)
    tf8 = jnp.broadcast_to(tf, (8, FREQ))
    u = jnp.dot(tf8, w1_ref[...], preferred_element_type=jnp.float32)
    u = u + b1_ref[...]
    u = u / (1.0 + jnp.exp(-u))
    te = jnp.dot(u, w2_ref[...], preferred_element_type=jnp.float32)
    te = te + b2_ref[...]
    o_ref[...] = bm_ref[...] + te


def _time_cvec(t_s, freqs, W1, b1r, W2, b2r, bm8):
    return pl.pallas_call(
        _time_body,
        out_shape=jax.ShapeDtypeStruct((8, H), jnp.float32),
        in_specs=[
            pl.BlockSpec(memory_space=pltpu.SMEM),
            pl.BlockSpec((1, FREQ // 2), lambda: (0, 0)),
            pl.BlockSpec((FREQ, H), lambda: (0, 0)),
            pl.BlockSpec((1, H), lambda: (0, 0)),
            pl.BlockSpec((H, H), lambda: (0, 0)),
            pl.BlockSpec((1, H), lambda: (0, 0)),
            pl.BlockSpec((8, H), lambda: (0, 0)),
        ],
        out_specs=pl.BlockSpec((8, H), lambda: (0, 0)),
    )(t_s, freqs, W1, b1r, W2, b2r, bm8)


def _rbf_body(d2_ref, mu_ref, wr_ref, cv_ref, r0, r1, r2, r3):
    d = jnp.sqrt(d2_ref[0] + 1e-8)
    g = d - mu_ref[...]
    e = jnp.exp(g * g * (-1.0 / 8.0))
    rr = lax.dot_general(e, wr_ref[...], (((0,), (0,)), ((), ())),
                         preferred_element_type=jnp.float32)
    cv = cv_ref[...]
    r0[...] = rr[:, 0 * H:1 * H] + cv[0:1]
    r1[...] = rr[:, 1 * H:2 * H] + cv[1:2]
    r2[...] = rr[:, 2 * H:3 * H] + cv[2:3]
    r3[...] = rr[:, 3 * H:4 * H] + cv[3:4]


def _rbf_project(d2m, mu, wr, cvec):
    rspec = pl.BlockSpec((COLS_D2, H), lambda i: (i, 0))
    return pl.pallas_call(
        _rbf_body,
        grid=(ROWS_D2,),
        out_shape=[jax.ShapeDtypeStruct((E_PAD, H), jnp.float32)] * NB,
        in_specs=[
            pl.BlockSpec((1, 1, COLS_D2), lambda i: (i, 0, 0)),
            pl.BlockSpec((NUM_RBF, 1), lambda i: (0, 0)),
            pl.BlockSpec((NUM_RBF, NB * H), lambda i: (0, 0)),
            pl.BlockSpec((8, H), lambda i: (0, 0)),
        ],
        out_specs=[rspec, rspec, rspec, rspec],
    )(d2m, mu, wr, cvec)


def _prep0_body(x_ref, w_ref, a_ref, b_ref):
    ab = jnp.dot(x_ref[...], w_ref[...], preferred_element_type=jnp.float32)
    a_ref[...] = ab[:, :H]
    b_ref[...] = ab[:, H:]


def _prep0(x, wsd):
    nspec = pl.BlockSpec((NBLK, H), lambda i: (i, 0))
    return pl.pallas_call(
        _prep0_body,
        grid=(NP // NBLK,),
        out_shape=[jax.ShapeDtypeStruct((NP, H), jnp.float32)] * 2,
        in_specs=[nspec, pl.BlockSpec((H, 2 * H), lambda i: (0, 0))],
        out_specs=[nspec, nspec],
    )(x, wsd)


def _update_body(last, h_ref, p_ref, wn1_ref, wn2_ref, bn_ref, wx_ref, bx_ref,
                 *outs):
    h = h_ref[...]
    agg = p_ref[0] + p_ref[1]
    u = (jnp.dot(h, wn1_ref[...], preferred_element_type=jnp.float32)
         + jnp.dot(agg, wn2_ref[...], preferred_element_type=jnp.float32)
         + bn_ref[...])
    hn = h + u / (1.0 + jnp.exp(-u))
    nx = jnp.dot(hn, wx_ref[...], preferred_element_type=jnp.float32)
    if last:
        outs[0][...] = nx + bx_ref[...]
    else:
        outs[0][...] = hn
        outs[1][...] = nx[:, :H]
        outs[2][...] = nx[:, H:]


def _update(h, p, wn1, wn2, bnr, wx, bxr, last):
    nspec = pl.BlockSpec((NBLK, H), lambda i: (i, 0))
    wspec = pl.BlockSpec((H, H), lambda i: (0, 0))
    bspec = pl.BlockSpec((1, H), lambda i: (0, 0))
    if last:
        out_shape = jax.ShapeDtypeStruct((NP, H), jnp.float32)
        out_specs = nspec
        wx_spec = pl.BlockSpec((H, H), lambda i: (0, 0))
    else:
        out_shape = [jax.ShapeDtypeStruct((NP, H), jnp.float32)] * 3
        out_specs = [nspec, nspec, nspec]
        wx_spec = pl.BlockSpec((H, 2 * H), lambda i: (0, 0))
    return pl.pallas_call(
        functools.partial(_update_body, last),
        grid=(NP // NBLK,),
        out_shape=out_shape,
        in_specs=[
            nspec,
            pl.BlockSpec((NC, NBLK, H), lambda i: (0, i, 0)),
            wspec, wspec, bspec, wx_spec, bspec,
        ],
        out_specs=out_specs,
    )(h, p, wn1, wn2, bnr, wx, bxr)


def kernel(x, pos, timestep, edge_index, W1, b1, W2, b2, Wm, bm, Wn, bn,
           Wout, bout):
    f32 = jnp.float32
    src = edge_index[0]
    dst = edge_index[1]
    px_a = pos[:, 0]
    py_a = pos[:, 1]
    pz_a = pos[:, 2]

    half = FREQ // 2
    freqs = jnp.exp(
        -math.log(10000.0) * jnp.arange(half, dtype=f32) / half
    ).reshape(1, half)
    mu = jnp.linspace(0.0, MAX_DIST, NUM_RBF).astype(f32).reshape(NUM_RBF, 1)
    wr = Wm[:, 2 * H:2 * H + NUM_RBF, :].transpose(1, 0, 2).reshape(
        NUM_RBF, NB * H)
    bm8 = jnp.concatenate([bm, jnp.zeros((8 - NB, H), f32)], axis=0)

    cvec = _time_cvec(timestep.reshape(1, 1), freqs, W1, b1.reshape(1, H),
                      W2, b2.reshape(1, H), bm8)

    d2 = _edge_dist2(px_a, py_a, pz_a, src, dst)
    d2p = jnp.pad(d2, (0, E_PAD - E)).reshape(ROWS_D2, 1, COLS_D2)
    rs = _rbf_project(d2p, mu, wr, cvec)

    srcp = jnp.pad(src, (0, E_PAD - E))
    dstp = jnp.pad(dst, (0, E_PAD - E), constant_values=NP - 1)

    zeros_n = jnp.zeros((NP, H), f32)
    wsd = [jnp.concatenate([Wm[i, :H, :], Wm[i, H:2 * H, :]], axis=1)
           for i in range(NB)]

    wout_pad = jnp.pad(Wout, ((0, 0), (0, H - OUT_TOKENS)))
    bout_pad = jnp.pad(bout, (0, H - OUT_TOKENS)).reshape(1, H)

    h = jnp.pad(x, ((0, NP - N), (0, 0)))
    a, b = _prep0(h, wsd[0])
    for i in range(NB):
        p = _edge_messages(a, b, rs[i], srcp, dstp, zeros_n)
        wn1 = Wn[i, :H, :]
        wn2 = Wn[i, H:, :]
        bnr = bn[i].reshape(1, H)
        if i < NB - 1:
            h, a, b = _update(h, p, wn1, wn2, bnr, wsd[i + 1], bnr, last=False)
        else:
            out128 = _update(h, p, wn1, wn2, bnr, wout_pad, bout_pad,
                             last=True)
    return out128[:N, :OUT_TOKENS]
